# baseline (speedup 1.0000x reference)
import functools

import jax
import jax.numpy as jnp
from jax import lax
from jax.experimental import pallas as pl
from jax.experimental.pallas import tpu as pltpu
from jax.experimental.pallas import tpu_sc as plsc

B, T, H = 4096, 200, 64
N = B * T
NC, NS = 2, 16
NW = NC * NS
BPW = B // NW
PER_W = BPW * T
HG, HS = H // 8, 8
BG, BL = B // 128, 128

_mesh = plsc.VectorSubcoreMesh(core_axis_name="c", subcore_axis_name="s")


def _iota16():
    return lax.iota(jnp.int32, 16)


VOCAB = 1000000
CB = VOCAB // 128
TAIL = VOCAB - CB * 128
BLK_PW = CB // NW + 1


@functools.partial(
    pl.kernel,
    mesh=_mesh,
    out_type=jax.ShapeDtypeStruct((VOCAB // 2, 128), jnp.float32),
    compiler_params=pltpu.CompilerParams(
        use_tc_tiling_on_sc=True, needs_layout_passes=False,
        disable_bounds_checks=True),
    scratch_types=[
        pltpu.VMEM((H, 128), jnp.float32),
        pltpu.VMEM((H, 128), jnp.float32),
        pltpu.VMEM((H, 128), jnp.float32),
        pltpu.VMEM((H, 128), jnp.float32),
        pltpu.VMEM((TAIL // 2, 128), jnp.float32),
        pltpu.SemaphoreType.DMA,
        pltpu.SemaphoreType.DMA,
        pltpu.SemaphoreType.DMA,
        pltpu.SemaphoreType.DMA,
    ],
)
def _wtrans_sc(wt_hbm, wtail_hbm, wc_hbm, src_a, src_b, dst_a, dst_b,
               tail_v, gsa, gsb, ssa, ssb):
    wid = lax.axis_index("s") * NC + lax.axis_index("c")

    iota = _iota16()
    civecs = [c0 + iota for c0 in range(0, 128, 16)]
    rvecs = [(c0 + iota) >> 1 for c0 in range(0, 128, 16)]
    lo64 = [((c0 + iota) & 1) * H for c0 in range(0, 128, 16)]

    def transpose_blk(src, dst):
        @plsc.parallel_loop(0, 16, unroll=8)
        def phase(p):
            shifts = (iota + p) & 15
            for h0 in range(0, H, 16):
                hvec = shifts + h0
                for k in range(8):
                    v = plsc.load_gather(src, [hvec, civecs[k]])
                    plsc.store_scatter(dst, [rvecs[k], lo64[k] + hvec], v)

    def gather_blk(src, sem, cg):
        return pltpu.async_copy(wt_hbm.at[:, pl.ds(cg * 128, 128)], src, sem)

    def put_blk(dst, sem, cg):
        return pltpu.async_copy(dst, wc_hbm.at[pl.ds(cg * H, H)], sem)

    def drain_blk(dst, sem, cg):
        pltpu.make_async_copy(dst, wc_hbm.at[pl.ds(cg * H, H)], sem).wait()

    def step(k, carry):
        cg0 = wid + (2 * k) * NW
        cg1 = cg0 + NW

        @pl.when((k > 0) & (cg0 < CB))
        def _():
            drain_blk(dst_a, ssa, cg0 - 2 * NW)

        @pl.when(cg0 < CB)
        def _():
            ga = gather_blk(src_a, gsa, cg0)

            @pl.when(k > 0)
            def _():
                drain_blk(dst_b, ssb, cg1 - 2 * NW)

            ga.wait()
            transpose_blk(src_a, dst_a)
            put_blk(dst_a, ssa, cg0)

        @pl.when(cg1 < CB)
        def _():
            gb = gather_blk(src_b, gsb, cg1)
            gb.wait()
            transpose_blk(src_b, dst_b)
            put_blk(dst_b, ssb, cg1)

        return carry

    nsteps = (BLK_PW + 1) // 2
    lax.fori_loop(0, nsteps, step, 0)

    last0 = wid + 2 * (nsteps - 1) * NW
    last1 = last0 + NW

    @pl.when(last0 < CB)
    def _():
        drain_blk(dst_a, ssa, last0)

    @pl.when(last1 < CB)
    def _():
        drain_blk(dst_b, ssb, last1)

    @pl.when(wid == 0)
    def _():
        pltpu.sync_copy(wtail_hbm, tail_v)
        pltpu.sync_copy(tail_v, wc_hbm.at[pl.ds(CB * H, TAIL // 2)])


@functools.partial(
    pl.kernel,
    mesh=_mesh,
    out_type=jax.ShapeDtypeStruct((T, HG, BG, HS, BL), jnp.float32),
    compiler_params=pltpu.CompilerParams(
        use_tc_tiling_on_sc=False, needs_layout_passes=False,
        disable_bounds_checks=True),
    scratch_types=[
        pltpu.VMEM((PER_W,), jnp.int32),
        pltpu.VMEM((T, BL), jnp.int32),
        pltpu.VMEM((BL, H), jnp.float32),
        pltpu.VMEM((BL, H), jnp.float32),
        pltpu.VMEM((H, BL), jnp.float32),
        pltpu.VMEM((H, BL), jnp.float32),
        pltpu.SemaphoreType.DMA,
        pltpu.SemaphoreType.DMA,
        pltpu.SemaphoreType.DMA,
        pltpu.SemaphoreType.DMA,
    ],
)
def _embed_sc(ids_hbm, w_hbm, out_hbm, idx_raw, idx_t, rows_a, rows_b,
              tr_a, tr_b, gsa, gsb, ssa, ssb):
    wid = lax.axis_index("s") * NC + lax.axis_index("c")
    base = wid * PER_W

    pltpu.sync_copy(ids_hbm.at[pl.ds(base, PER_W)], idx_raw)

    iota = _iota16()
    rowvecs = [b0 + iota for b0 in range(0, BL, 16)]
    idsvecs = [(b0 + iota) * T for b0 in range(0, BL, 16)]

    def transpose_ids(t, carry):
        for k, bv in enumerate(idsvecs):
            v = plsc.load_gather(idx_raw, [bv + t])
            idx_t[t, pl.ds(k * 16, 16)] = v
        return carry

    lax.fori_loop(0, T, transpose_ids, 0)

    def transpose_tile(rows, tr):
        @plsc.parallel_loop(0, 16, unroll=8)
        def phase(p):
            shifts = (iota + p) & 15
            for h0 in range(0, H, 16):
                hvec = shifts + h0
                for rv in rowvecs:
                    v = plsc.load_gather(rows, [rv, hvec])
                    plsc.store_scatter(tr, [hvec, rv], v)

    def write_out(tr, t, sem):
        for hg in range(HG):
            pltpu.async_copy(
                tr.at[pl.ds(hg * HS, HS)], out_hbm.at[t, hg, wid], sem)

    def drain_out(tr, t, sem):
        for hg in range(HG):
            pltpu.make_async_copy(
                tr.at[pl.ds(hg * HS, HS)], out_hbm.at[t, hg, wid], sem).wait()

    def step(tt, carry):
        t0 = tt * 2
        t1 = t0 + 1

        @pl.when(tt > 0)
        def _():
            drain_out(tr_a, t0 - 2, ssa)

        ga = pltpu.async_copy(w_hbm.at[idx_t.at[t0]], rows_a, gsa)

        @pl.when(tt > 0)
        def _():
            drain_out(tr_b, t1 - 2, ssb)

        gb = pltpu.async_copy(w_hbm.at[idx_t.at[t1]], rows_b, gsb)

        ga.wait()
        transpose_tile(rows_a, tr_a)
        write_out(tr_a, t0, ssa)

        gb.wait()
        transpose_tile(rows_b, tr_b)
        write_out(tr_b, t1, ssb)
        return carry

    lax.fori_loop(0, T // 2, step, 0)

    drain_out(tr_a, T - 2, ssa)
    drain_out(tr_b, T - 1, ssb)


def kernel(input_ids, weight):
    ids = input_ids.astype(jnp.int32).reshape(N)
    wt = weight.T
    wtail = weight[CB * 128:].reshape(TAIL // 2, 128)
    wc = _wtrans_sc(wt, wtail)
    w_lin = wc.reshape(VOCAB, H)
    out5d = _embed_sc(ids, w_lin)
    return out5d.transpose(2, 4, 0, 1, 3).reshape(B, T, H)


# speedup vs baseline: 1.2336x; 1.2336x over previous
import functools

import jax
import jax.numpy as jnp
from jax import lax
from jax.experimental import pallas as pl
from jax.experimental.pallas import tpu as pltpu
from jax.experimental.pallas import tpu_sc as plsc

B, T, H = 4096, 200, 64
N = B * T
NC, NS = 2, 16
NW = NC * NS
BPW = B // NW
PER_W = BPW * T
HG, HS = H // 8, 8
BG, BL = B // 128, 128

_mesh = plsc.VectorSubcoreMesh(core_axis_name="c", subcore_axis_name="s")


def _iota16():
    return lax.iota(jnp.int32, 16)


VOCAB = 1000000
CB = VOCAB // 128
TAIL = VOCAB - CB * 128
BLK_PW = CB // NW + 1


@functools.partial(
    pl.kernel,
    mesh=_mesh,
    out_type=jax.ShapeDtypeStruct((VOCAB // 2, 128), jnp.float32),
    compiler_params=pltpu.CompilerParams(
        use_tc_tiling_on_sc=True, needs_layout_passes=False,
        disable_bounds_checks=True),
    scratch_types=[
        pltpu.VMEM((H, 128), jnp.float32),
        pltpu.VMEM((H, 128), jnp.float32),
        pltpu.VMEM((H, 128), jnp.float32),
        pltpu.VMEM((H, 128), jnp.float32),
        pltpu.VMEM((TAIL // 2, 128), jnp.float32),
        pltpu.SemaphoreType.DMA,
        pltpu.SemaphoreType.DMA,
        pltpu.SemaphoreType.DMA,
        pltpu.SemaphoreType.DMA,
    ],
)
def _wtrans_sc(wt_hbm, wtail_hbm, wc_hbm, src_a, src_b, dst_a, dst_b,
               tail_v, gsa, gsb, ssa, ssb):
    wid = lax.axis_index("s") * NC + lax.axis_index("c")

    iota = _iota16()
    civecs = [c0 + iota for c0 in range(0, 128, 16)]
    rvecs = [(c0 + iota) >> 1 for c0 in range(0, 128, 16)]
    lo64 = [((c0 + iota) & 1) * H for c0 in range(0, 128, 16)]

    def transpose_blk(src, dst):
        @plsc.parallel_loop(0, 16, unroll=4)
        def phase(p):
            shifts = (iota + p) & 15
            for h0 in range(0, H, 16):
                hvec = shifts + h0
                for k in range(8):
                    v = plsc.load_gather(src, [hvec, civecs[k]])
                    plsc.store_scatter(dst, [rvecs[k], lo64[k] + hvec], v)

    def gather_blk(src, sem, cg):
        return pltpu.async_copy(wt_hbm.at[:, pl.ds(cg * 128, 128)], src, sem)

    def put_blk(dst, sem, cg):
        return pltpu.async_copy(dst, wc_hbm.at[pl.ds(cg * H, H)], sem)

    def drain_blk(dst, sem, cg):
        pltpu.make_async_copy(dst, wc_hbm.at[pl.ds(cg * H, H)], sem).wait()

    def step(k, carry):
        cg0 = wid + (2 * k) * NW
        cg1 = cg0 + NW

        @pl.when(cg0 < CB)
        def _():
            @pl.when(k > 0)
            def _():
                drain_blk(dst_a, ssa, cg0 - 2 * NW)

            gather_blk(src_a, gsa, cg0)

        @pl.when(cg1 < CB)
        def _():
            @pl.when(k > 0)
            def _():
                drain_blk(dst_b, ssb, cg1 - 2 * NW)

            gather_blk(src_b, gsb, cg1)

        @pl.when(cg0 < CB)
        def _():
            pltpu.make_async_copy(
                wt_hbm.at[:, pl.ds(cg0 * 128, 128)], src_a, gsa).wait()
            transpose_blk(src_a, dst_a)
            put_blk(dst_a, ssa, cg0)

        @pl.when(cg1 < CB)
        def _():
            pltpu.make_async_copy(
                wt_hbm.at[:, pl.ds(cg1 * 128, 128)], src_b, gsb).wait()
            transpose_blk(src_b, dst_b)
            put_blk(dst_b, ssb, cg1)

        return carry

    nsteps = (BLK_PW + 1) // 2
    lax.fori_loop(0, nsteps, step, 0)

    last0 = wid + 2 * (nsteps - 1) * NW
    last1 = last0 + NW

    @pl.when(last0 < CB)
    def _():
        drain_blk(dst_a, ssa, last0)

    @pl.when(last1 < CB)
    def _():
        drain_blk(dst_b, ssb, last1)

    @pl.when(wid == 0)
    def _():
        pltpu.sync_copy(wtail_hbm, tail_v)
        pltpu.sync_copy(tail_v, wc_hbm.at[pl.ds(CB * H, TAIL // 2)])


@functools.partial(
    pl.kernel,
    mesh=_mesh,
    out_type=jax.ShapeDtypeStruct((T, HG, BG, HS, BL), jnp.float32),
    compiler_params=pltpu.CompilerParams(
        use_tc_tiling_on_sc=False, needs_layout_passes=False,
        disable_bounds_checks=True),
    scratch_types=[
        pltpu.VMEM((PER_W,), jnp.int32),
        pltpu.VMEM((T, BL), jnp.int32),
        pltpu.VMEM((BL, H), jnp.float32),
        pltpu.VMEM((BL, H), jnp.float32),
        pltpu.VMEM((H, BL), jnp.float32),
        pltpu.VMEM((H, BL), jnp.float32),
        pltpu.SemaphoreType.DMA,
        pltpu.SemaphoreType.DMA,
        pltpu.SemaphoreType.DMA,
        pltpu.SemaphoreType.DMA,
    ],
)
def _embed_sc(ids_hbm, w_hbm, out_hbm, idx_raw, idx_t, rows_a, rows_b,
              tr_a, tr_b, gsa, gsb, ssa, ssb):
    wid = lax.axis_index("s") * NC + lax.axis_index("c")
    base = wid * PER_W

    pltpu.sync_copy(ids_hbm.at[pl.ds(base, PER_W)], idx_raw)

    iota = _iota16()
    rowvecs = [b0 + iota for b0 in range(0, BL, 16)]
    idsvecs = [(b0 + iota) * T for b0 in range(0, BL, 16)]

    def transpose_ids(t, carry):
        for k, bv in enumerate(idsvecs):
            v = plsc.load_gather(idx_raw, [bv + t])
            idx_t[t, pl.ds(k * 16, 16)] = v
        return carry

    lax.fori_loop(0, T, transpose_ids, 0)

    def transpose_tile(rows, tr):
        @plsc.parallel_loop(0, 16, unroll=4)
        def phase(p):
            shifts = (iota + p) & 15
            for h0 in range(0, H, 16):
                hvec = shifts + h0
                for rv in rowvecs:
                    v = plsc.load_gather(rows, [rv, hvec])
                    plsc.store_scatter(tr, [hvec, rv], v)

    def write_out(tr, t, sem):
        for hg in range(HG):
            pltpu.async_copy(
                tr.at[pl.ds(hg * HS, HS)], out_hbm.at[t, hg, wid], sem)

    def drain_out(tr, t, sem):
        for hg in range(HG):
            pltpu.make_async_copy(
                tr.at[pl.ds(hg * HS, HS)], out_hbm.at[t, hg, wid], sem).wait()

    def step(tt, carry):
        t0 = tt * 2
        t1 = t0 + 1

        @pl.when(tt > 0)
        def _():
            drain_out(tr_a, t0 - 2, ssa)

        ga = pltpu.async_copy(w_hbm.at[idx_t.at[t0]], rows_a, gsa)

        @pl.when(tt > 0)
        def _():
            drain_out(tr_b, t1 - 2, ssb)

        gb = pltpu.async_copy(w_hbm.at[idx_t.at[t1]], rows_b, gsb)

        ga.wait()
        transpose_tile(rows_a, tr_a)
        write_out(tr_a, t0, ssa)

        gb.wait()
        transpose_tile(rows_b, tr_b)
        write_out(tr_b, t1, ssb)
        return carry

    lax.fori_loop(0, T // 2, step, 0)

    drain_out(tr_a, T - 2, ssa)
    drain_out(tr_b, T - 1, ssb)


def kernel(input_ids, weight):
    ids = input_ids.astype(jnp.int32).reshape(N)
    wt = weight.T
    wtail = weight[CB * 128:].reshape(TAIL // 2, 128)
    wc = _wtrans_sc(wt, wtail)
    w_lin = wc.reshape(VOCAB, H)
    out5d = _embed_sc(ids, w_lin)
    return out5d.transpose(2, 4, 0, 1, 3).reshape(B, T, H)
